# Initial kernel scaffold; baseline (speedup 1.0000x reference)
import jax
import jax.numpy as jnp
from jax.experimental import pallas as pl


def kernel(word_ids, extword_ids, word_table, ext_table):
    raise NotImplementedError("write your pallas kernel here")


# speedup vs baseline: 6.2056x; 6.2056x over previous
import functools

import jax
import jax.numpy as jnp
from jax import lax
from jax.experimental import pallas as pl
from jax.experimental.pallas import tpu as pltpu
from jax.experimental.pallas import tpu_sc as plsc

DIM = 128
CHUNK = 128
LANES = 16


@functools.lru_cache(maxsize=None)
def _build(total):
    info = plsc.get_sparse_core_info()
    nc, ns = info.num_cores, info.num_subcores
    nw = nc * ns
    assert total % (nw * CHUNK) == 0
    b_per_w = total // nw
    n_chunks = b_per_w // CHUNK

    mesh = plsc.VectorSubcoreMesh(core_axis_name="c", subcore_axis_name="s")

    @functools.partial(
        pl.kernel,
        mesh=mesh,
        out_type=jax.ShapeDtypeStruct((total, DIM), jnp.float32),
        scratch_types=[
            pltpu.VMEM((CHUNK,), jnp.int32),
            pltpu.VMEM((CHUNK,), jnp.int32),
            pltpu.VMEM((CHUNK, DIM), jnp.float32),
            pltpu.VMEM((CHUNK, DIM), jnp.float32),
            pltpu.SemaphoreType.DMA,
            pltpu.SemaphoreType.DMA,
        ],
    )
    def emb_kernel(w_ids, e_ids, w_tab, e_tab, out, idx1, idx2, buf1, buf2,
                   sem1, sem2):
        wid = lax.axis_index("s") * nc + lax.axis_index("c")
        base = wid * b_per_w

        def chunk_body(i, carry):
            off = base + i * CHUNK
            pltpu.sync_copy(w_ids.at[pl.ds(off, CHUNK)], idx1)
            pltpu.sync_copy(e_ids.at[pl.ds(off, CHUNK)], idx2)
            cp1 = pltpu.async_copy(w_tab.at[idx1], buf1, sem1)
            cp2 = pltpu.async_copy(e_tab.at[idx2], buf2, sem2)
            cp1.wait()
            cp2.wait()

            def row_body(r, c2):
                for g in range(DIM // LANES):
                    sl = pl.ds(g * LANES, LANES)
                    v = buf2[r, sl]
                    plsc.addupdate(buf1.at[r, sl], v)
                return c2

            lax.fori_loop(0, CHUNK, row_body, 0)
            pltpu.sync_copy(buf1, out.at[pl.ds(off, CHUNK)])
            return carry

        lax.fori_loop(0, n_chunks, chunk_body, 0)

    return emb_kernel


def kernel(word_ids, extword_ids, word_table, ext_table):
    b, l = word_ids.shape
    total = b * l
    w_flat = word_ids.reshape(total).astype(jnp.int32)
    e_flat = extword_ids.reshape(total).astype(jnp.int32)
    out = _build(total)(w_flat, e_flat, word_table, ext_table)
    return out.reshape(b, l, DIM)


# speedup vs baseline: 12.9372x; 2.0848x over previous
import functools

import jax
import jax.numpy as jnp
from jax import lax
from jax.experimental import pallas as pl
from jax.experimental.pallas import tpu as pltpu
from jax.experimental.pallas import tpu_sc as plsc

DIM = 128
CHUNK = 128
LANES = 16
QBLK = 40


@functools.lru_cache(maxsize=None)
def _build(total):
    info = plsc.get_sparse_core_info()
    nc, ns = info.num_cores, info.num_subcores
    nw = nc * ns
    b_per_w = total // nw
    n_chunks = b_per_w // CHUNK
    assert total % (nw * CHUNK) == 0 and n_chunks % QBLK == 0 and n_chunks % 2 == 0

    mesh = plsc.VectorSubcoreMesh(core_axis_name="c", subcore_axis_name="s")

    @functools.partial(
        pl.kernel,
        mesh=mesh,
        out_type=jax.ShapeDtypeStruct((total, DIM), jnp.float32),
        scratch_types=[
            pltpu.VMEM((2, QBLK, CHUNK), jnp.int32),
            pltpu.VMEM((2, QBLK, CHUNK), jnp.int32),
            pltpu.VMEM((CHUNK, DIM), jnp.float32),
            pltpu.VMEM((CHUNK, DIM), jnp.float32),
            pltpu.VMEM((CHUNK, DIM), jnp.float32),
            pltpu.VMEM((CHUNK, DIM), jnp.float32),
            pltpu.VMEM((CHUNK, DIM), jnp.float32),
            pltpu.VMEM((CHUNK, DIM), jnp.float32),
            pltpu.SemaphoreType.DMA,
            pltpu.SemaphoreType.DMA,
            pltpu.SemaphoreType.DMA,
            pltpu.SemaphoreType.DMA,
            pltpu.SemaphoreType.DMA,
            pltpu.SemaphoreType.DMA,
        ],
    )
    def emb_kernel(w_ids, e_ids, w_tab, e_tab, out,
                   idxw, idxe, g1a, g2a, g1b, g2b, oba, obb,
                   gws_a, ges_a, gws_b, ges_b, ws_a, ws_b):
        wid = lax.axis_index("s") * nc + lax.axis_index("c")
        cbase = wid * n_chunks

        sets = ((g1a, g2a, gws_a, ges_a), (g1b, g2b, gws_b, ges_b))
        obufs = ((oba, ws_a), (obb, ws_b))

        def load_idx(q):
            slot = lax.rem(q, 2)
            src = pl.ds(cbase + q * QBLK, QBLK)
            pltpu.sync_copy(w_ids.at[src], idxw.at[slot])
            pltpu.sync_copy(e_ids.at[src], idxe.at[slot])

        def issue_gather(i, b):
            g1, g2, gws, ges = sets[b]
            q = lax.div(i, QBLK)
            slot = lax.rem(q, 2)
            row = lax.rem(i, QBLK)
            pltpu.async_copy(w_tab.at[idxw.at[slot, row]], g1, gws)
            pltpu.async_copy(e_tab.at[idxe.at[slot, row]], g2, ges)

        load_idx(0)
        issue_gather(0, 0)
        issue_gather(1, 1)

        def outer(i2, carry):
            for b in range(2):
                g1, g2, gws, ges = sets[b]
                ob, ws = obufs[b]
                i = 2 * i2 + b
                pltpu.make_async_copy(w_tab.at[pl.ds(0, CHUNK)], g1, gws).wait()
                pltpu.make_async_copy(w_tab.at[pl.ds(0, CHUNK)], g2, ges).wait()

                @pl.when(i2 >= 1)
                def _():
                    pltpu.make_async_copy(w_tab.at[pl.ds(0, CHUNK)], ob, ws).wait()

                def row_body(r, c):
                    for g in range(DIM // LANES):
                        sl = pl.ds(g * LANES, LANES)
                        ob[r, sl] = g1[r, sl] + g2[r, sl]
                    return c

                lax.fori_loop(0, CHUNK, row_body, 0)

                pltpu.async_copy(ob, out.at[pl.ds((cbase + i) * CHUNK, CHUNK)], ws)

                nxt = i + 2

                @pl.when(nxt < n_chunks)
                def _():
                    @pl.when(lax.rem(nxt, QBLK) == 0)
                    def _():
                        load_idx(lax.div(nxt, QBLK))

                    issue_gather(nxt, b)
            return carry

        lax.fori_loop(0, n_chunks // 2, outer, 0)
        pltpu.make_async_copy(w_tab.at[pl.ds(0, CHUNK)], oba, ws_a).wait()
        pltpu.make_async_copy(w_tab.at[pl.ds(0, CHUNK)], obb, ws_b).wait()

    return emb_kernel


def kernel(word_ids, extword_ids, word_table, ext_table):
    b, l = word_ids.shape
    total = b * l
    w_2d = word_ids.reshape(total // CHUNK, CHUNK).astype(jnp.int32)
    e_2d = extword_ids.reshape(total // CHUNK, CHUNK).astype(jnp.int32)
    out = _build(total)(w_2d, e_2d, word_table, ext_table)
    return out.reshape(b, l, DIM)


# speedup vs baseline: 12.9641x; 1.0021x over previous
import functools

import jax
import jax.numpy as jnp
from jax import lax
from jax.experimental import pallas as pl
from jax.experimental.pallas import tpu as pltpu
from jax.experimental.pallas import tpu_sc as plsc

DIM = 128
CHUNK = 128
LANES = 16
QBLK = 40


@functools.lru_cache(maxsize=None)
def _build(total):
    info = plsc.get_sparse_core_info()
    nc, ns = info.num_cores, info.num_subcores
    nw = nc * ns
    b_per_w = total // nw
    n_chunks = b_per_w // CHUNK
    assert total % (nw * CHUNK) == 0 and n_chunks % QBLK == 0 and n_chunks % 4 == 0

    mesh = plsc.VectorSubcoreMesh(core_axis_name="c", subcore_axis_name="s")

    @functools.partial(
        pl.kernel,
        mesh=mesh,
        out_type=jax.ShapeDtypeStruct((total, DIM), jnp.float32),
        scratch_types=[
            pltpu.VMEM((2, QBLK, CHUNK), jnp.int32),
            pltpu.VMEM((2, QBLK, CHUNK), jnp.int32),
            pltpu.VMEM((CHUNK, DIM), jnp.float32),
            pltpu.VMEM((CHUNK, DIM), jnp.float32),
            pltpu.VMEM((CHUNK, DIM), jnp.float32),
            pltpu.VMEM((CHUNK, DIM), jnp.float32),
            pltpu.VMEM((CHUNK, DIM), jnp.float32),
            pltpu.VMEM((CHUNK, DIM), jnp.float32),
            pltpu.SemaphoreType.DMA,
            pltpu.SemaphoreType.DMA,
            pltpu.SemaphoreType.DMA,
            pltpu.SemaphoreType.DMA,
            pltpu.SemaphoreType.DMA,
            pltpu.SemaphoreType.DMA,
            pltpu.SemaphoreType.DMA,
            pltpu.SemaphoreType.DMA,
            pltpu.SemaphoreType.DMA,
            pltpu.SemaphoreType.DMA,
        ],
    )
    def emb_kernel(w_ids, e_ids, w_tab, e_tab, out,
                   idxw, idxe, g1_0, g1_1, g1_2, g1_3, g2_0, g2_1,
                   gws_0, gws_1, gws_2, gws_3, ges_0, ges_1,
                   ws_0, ws_1, ws_2, ws_3):
        wid = lax.axis_index("s") * nc + lax.axis_index("c")
        cbase = wid * n_chunks

        g1s = (g1_0, g1_1, g1_2, g1_3)
        gwss = (gws_0, gws_1, gws_2, gws_3)
        wss = (ws_0, ws_1, ws_2, ws_3)
        g2s = (g2_0, g2_1)
        gess = (ges_0, ges_1)

        def drain(sem, buf):
            pltpu.make_async_copy(w_tab.at[pl.ds(0, CHUNK)], buf, sem).wait()

        def load_idx(q):
            slot = lax.rem(q, 2)
            src = pl.ds(cbase + q * QBLK, QBLK)
            pltpu.sync_copy(w_ids.at[src], idxw.at[slot])
            pltpu.sync_copy(e_ids.at[src], idxe.at[slot])

        def issue_gather(i, a, b):
            q = lax.div(i, QBLK)
            slot = lax.rem(q, 2)
            row = lax.rem(i, QBLK)
            pltpu.async_copy(w_tab.at[idxw.at[slot, row]], g1s[a], gwss[a])
            pltpu.async_copy(e_tab.at[idxe.at[slot, row]], g2s[b], gess[b])

        load_idx(0)
        issue_gather(0, 0, 0)
        issue_gather(1, 1, 1)

        def outer(i2, carry):
            for b4 in range(4):
                b2 = b4 % 2
                i = 4 * i2 + b4
                g1, g2 = g1s[b4], g2s[b2]

                drain(gwss[b4], g1)
                drain(gess[b2], g2)

                def row_body(r, c):
                    for g in range(DIM // LANES):
                        sl = pl.ds(g * LANES, LANES)
                        plsc.addupdate(g1.at[r, sl], g2[r, sl])
                    return c

                lax.fori_loop(0, CHUNK, row_body, 0)

                pltpu.async_copy(g1, out.at[pl.ds((cbase + i) * CHUNK, CHUNK)],
                                 wss[b4])

                nxt = i + 2
                na = (b4 + 2) % 4

                def prefetch():
                    @pl.when(lax.rem(nxt, QBLK) == 0)
                    def _():
                        load_idx(lax.div(nxt, QBLK))

                    issue_gather(nxt, na, b2)

                if b4 >= 2:
                    @pl.when(nxt < n_chunks)
                    def _():
                        drain(wss[na], g1s[na])
                        prefetch()
                else:
                    @pl.when(nxt < n_chunks)
                    def _():
                        @pl.when(i2 >= 1)
                        def _():
                            drain(wss[na], g1s[na])

                        prefetch()
            return carry

        lax.fori_loop(0, n_chunks // 4, outer, 0)
        for a in range(4):
            drain(wss[a], g1s[a])

    return emb_kernel


def kernel(word_ids, extword_ids, word_table, ext_table):
    b, l = word_ids.shape
    total = b * l
    w_2d = word_ids.reshape(total // CHUNK, CHUNK).astype(jnp.int32)
    e_2d = extword_ids.reshape(total // CHUNK, CHUNK).astype(jnp.int32)
    out = _build(total)(w_2d, e_2d, word_table, ext_table)
    return out.reshape(b, l, DIM)


# speedup vs baseline: 13.1390x; 1.0135x over previous
import functools

import jax
import jax.numpy as jnp
from jax import lax
from jax.experimental import pallas as pl
from jax.experimental.pallas import tpu as pltpu
from jax.experimental.pallas import tpu_sc as plsc

DIM = 128
CHUNK = 128
LANES = 16
QBLK = 40


@functools.lru_cache(maxsize=None)
def _build(total):
    info = plsc.get_sparse_core_info()
    nc, ns = info.num_cores, info.num_subcores
    nw = nc * ns
    b_per_w = total // nw
    n_chunks = b_per_w // CHUNK
    assert total % (nw * CHUNK) == 0 and n_chunks % QBLK == 0 and n_chunks % 4 == 0

    mesh = plsc.VectorSubcoreMesh(core_axis_name="c", subcore_axis_name="s")

    @functools.partial(
        pl.kernel,
        mesh=mesh,
        out_type=jax.ShapeDtypeStruct((total, DIM), jnp.float32),
        scratch_types=[
            pltpu.VMEM((3, QBLK, CHUNK), jnp.int32),
            pltpu.VMEM((3, QBLK, CHUNK), jnp.int32),
            pltpu.VMEM((CHUNK, DIM), jnp.float32),
            pltpu.VMEM((CHUNK, DIM), jnp.float32),
            pltpu.VMEM((CHUNK, DIM), jnp.float32),
            pltpu.VMEM((CHUNK, DIM), jnp.float32),
            pltpu.VMEM((CHUNK, DIM), jnp.float32),
            pltpu.VMEM((CHUNK, DIM), jnp.float32),
            pltpu.SemaphoreType.DMA,
            pltpu.SemaphoreType.DMA,
            pltpu.SemaphoreType.DMA,
            pltpu.SemaphoreType.DMA,
            pltpu.SemaphoreType.DMA,
            pltpu.SemaphoreType.DMA,
            pltpu.SemaphoreType.DMA,
            pltpu.SemaphoreType.DMA,
            pltpu.SemaphoreType.DMA,
            pltpu.SemaphoreType.DMA,
            pltpu.SemaphoreType.DMA,
            pltpu.SemaphoreType.DMA,
        ],
    )
    def emb_kernel(w_ids, e_ids, w_tab, e_tab, out,
                   idxw, idxe, g1_0, g1_1, g1_2, g1_3, g2_0, g2_1,
                   gws_0, gws_1, gws_2, gws_3, ges_0, ges_1,
                   ws_0, ws_1, ws_2, ws_3, ixw_sem, ixe_sem):
        wid = lax.axis_index("s") * nc + lax.axis_index("c")
        cbase = wid * n_chunks

        g1s = (g1_0, g1_1, g1_2, g1_3)
        gwss = (gws_0, gws_1, gws_2, gws_3)
        wss = (ws_0, ws_1, ws_2, ws_3)
        g2s = (g2_0, g2_1)
        gess = (ges_0, ges_1)

        def drain(sem, buf):
            pltpu.make_async_copy(w_tab.at[pl.ds(0, CHUNK)], buf, sem).wait()

        n_blocks = n_chunks // QBLK

        def start_load_idx(q):
            slot = lax.rem(q, 3)
            src = pl.ds(cbase + q * QBLK, QBLK)
            pltpu.async_copy(w_ids.at[src], idxw.at[slot], ixw_sem)
            pltpu.async_copy(e_ids.at[src], idxe.at[slot], ixe_sem)

        def wait_load_idx():
            pltpu.make_async_copy(w_ids.at[pl.ds(0, QBLK)], idxw.at[0],
                                  ixw_sem).wait()
            pltpu.make_async_copy(e_ids.at[pl.ds(0, QBLK)], idxe.at[0],
                                  ixe_sem).wait()

        def issue_gather(i, a, b):
            q = lax.div(i, QBLK)
            slot = lax.rem(q, 3)
            row = lax.rem(i, QBLK)
            pltpu.async_copy(w_tab.at[idxw.at[slot, row]], g1s[a], gwss[a])
            pltpu.async_copy(e_tab.at[idxe.at[slot, row]], g2s[b], gess[b])

        start_load_idx(0)
        wait_load_idx()
        start_load_idx(1)
        issue_gather(0, 0, 0)
        issue_gather(1, 1, 1)

        def outer(i2, carry):
            for b4 in range(4):
                b2 = b4 % 2
                i = 4 * i2 + b4
                g1, g2 = g1s[b4], g2s[b2]

                drain(gwss[b4], g1)
                drain(gess[b2], g2)

                def row_body(r, c):
                    for g in range(DIM // LANES):
                        sl = pl.ds(g * LANES, LANES)
                        plsc.addupdate(g1.at[r, sl], g2[r, sl])
                    return c

                lax.fori_loop(0, CHUNK, row_body, 0)

                pltpu.async_copy(g1, out.at[pl.ds((cbase + i) * CHUNK, CHUNK)],
                                 wss[b4])

                nxt = i + 2
                na = (b4 + 2) % 4

                def prefetch():
                    @pl.when(lax.rem(nxt, QBLK) == 0)
                    def _():
                        wait_load_idx()

                        @pl.when(lax.div(nxt, QBLK) + 1 < n_blocks)
                        def _():
                            start_load_idx(lax.div(nxt, QBLK) + 1)

                    issue_gather(nxt, na, b2)

                if b4 >= 2:
                    @pl.when(nxt < n_chunks)
                    def _():
                        drain(wss[na], g1s[na])
                        prefetch()
                else:
                    @pl.when(nxt < n_chunks)
                    def _():
                        @pl.when(i2 >= 1)
                        def _():
                            drain(wss[na], g1s[na])

                        prefetch()
            return carry

        lax.fori_loop(0, n_chunks // 4, outer, 0)
        for a in range(4):
            drain(wss[a], g1s[a])

    return emb_kernel


def kernel(word_ids, extword_ids, word_table, ext_table):
    b, l = word_ids.shape
    total = b * l
    w_2d = word_ids.reshape(total // CHUNK, CHUNK).astype(jnp.int32)
    e_2d = extword_ids.reshape(total // CHUNK, CHUNK).astype(jnp.int32)
    out = _build(total)(w_2d, e_2d, word_table, ext_table)
    return out.reshape(b, l, DIM)
